# baseline (speedup 1.0000x reference)
import functools

import jax
import jax.numpy as jnp
from jax import lax
from jax.experimental import pallas as pl
from jax.experimental.pallas import tpu as pltpu
from jax.experimental.pallas import tpu_sc as plsc

_N = 100000
_Z = 64
_R = 16
_B = 4096
_P = 2 * _Z

_NW = 32
_BPW = _B // _NW
_CH = 64


def _sc_gather(y, y2, lr2, mu2, dg2):
    mesh = plsc.VectorSubcoreMesh(core_axis_name="c", subcore_axis_name="s")

    @functools.partial(
        pl.kernel,
        mesh=mesh,
        out_type=(
            jax.ShapeDtypeStruct((_B, _Z * _R), jnp.float32),
            jax.ShapeDtypeStruct((_B, _P), jnp.float32),
            jax.ShapeDtypeStruct((_B, _P), jnp.float32),
        ),
        scratch_types=[
            pltpu.VMEM((_BPW,), jnp.int32),
            pltpu.VMEM((_BPW,), jnp.int32),
            pltpu.VMEM((_CH, _Z * _R), jnp.float32),
            pltpu.VMEM((_BPW, _P), jnp.float32),
            pltpu.VMEM((_BPW, _P), jnp.float32),
            pltpu.SemaphoreType.DMA,
            pltpu.SemaphoreType.DMA,
            pltpu.SemaphoreType.DMA,
        ],
    )
    def k(y_hbm, y2_hbm, lr_hbm, mu_hbm, dg_hbm, lrg_hbm, mug_hbm, dgg_hbm,
          idx_v, idx2_v, rows_v, mu_v, dg_v, sem_lr, sem_mu, sem_dg):
        wid = lax.axis_index("s") * 2 + lax.axis_index("c")
        base = wid * _BPW
        pltpu.sync_copy(y_hbm.at[pl.ds(base, _BPW)], idx_v)
        pltpu.sync_copy(y2_hbm.at[pl.ds(base, _BPW)], idx2_v)
        cp_mu = pltpu.async_copy(mu_hbm.at[idx2_v], mu_v, sem_mu)
        cp_dg = pltpu.async_copy(dg_hbm.at[idx2_v], dg_v, sem_dg)
        for c in range(_BPW // _CH):
            idx_c = idx_v.at[pl.ds(c * _CH, _CH)]
            pltpu.async_copy(lr_hbm.at[idx_c], rows_v, sem_lr).wait()
            pltpu.sync_copy(rows_v, lrg_hbm.at[pl.ds(base + c * _CH, _CH)])
        cp_mu.wait()
        pltpu.sync_copy(mu_v, mug_hbm.at[pl.ds(base, _BPW)])
        cp_dg.wait()
        pltpu.sync_copy(dg_v, dgg_hbm.at[pl.ds(base, _BPW)])

    return k(y, y2, lr2, mu2, dg2)


_BC = 256


def _tc_body(lrg_ref, mug_ref, dgg_ref, par_ref, out_ref):
    gt = lrg_ref[...].T
    gt3 = gt.reshape(_Z, _R, _BC)
    par = par_ref[...]
    mut = mug_ref[...].T
    dgt = dgg_ref[...].T
    mu_t = mut[0:_Z] + par * (mut[_Z:_P] - mut[0:_Z])
    dg_t = dgt[0:_Z] + par * (dgt[_Z:_P] - dgt[0:_Z])
    sp = jax.nn.softplus(dg_t)
    out_ref[0] = mu_t
    for j in range(_Z):
        if j > 0:
            out_ref[1 + j, 0:j] = jnp.zeros((j, _BC), jnp.float32)
        out_ref[1 + j, j:j + 1] = sp[j:j + 1]
        if j < _Z - 1:
            pj = gt3[j]
            prod = gt3[j + 1:] * pj[None]
            out_ref[1 + j, j + 1:_Z] = prod.sum(axis=1)


def _tc_build(lrg, mug, dgg, par):
    return pl.pallas_call(
        _tc_body,
        grid=(_B // _BC,),
        in_specs=[
            pl.BlockSpec((_BC, _Z * _R), lambda g: (g, 0)),
            pl.BlockSpec((_BC, _P), lambda g: (g, 0)),
            pl.BlockSpec((_BC, _P), lambda g: (g, 0)),
            pl.BlockSpec((1, _BC), lambda g: (0, g)),
        ],
        out_specs=pl.BlockSpec((_Z + 1, _Z, _BC), lambda g: (0, 0, g)),
        out_shape=jax.ShapeDtypeStruct((_Z + 1, _Z, _B), jnp.float32),
    )(lrg, mug, dgg, par)


def kernel(y, mu, low_rank, diag):
    lr2 = low_rank.reshape(_N, _Z * _R)
    mu2 = mu.reshape(_N // 2, _P)
    dg2 = diag.reshape(_N // 2, _P)
    y2 = lax.div(y, 2)
    par = (y % 2).astype(jnp.float32).reshape(1, _B)
    lrg, mug, dgg = _sc_gather(y, y2, lr2, mu2, dg2)
    out_t = _tc_build(lrg, mug, dgg, par)
    return jnp.transpose(out_t, (2, 1, 0))


# speedup vs baseline: 1.1657x; 1.1657x over previous
import functools

import jax
import jax.numpy as jnp
from jax import lax
from jax.experimental import pallas as pl
from jax.experimental.pallas import tpu as pltpu
from jax.experimental.pallas import tpu_sc as plsc

_N = 100000
_Z = 64
_R = 16
_B = 4096
_P = 2 * _Z

_NW = 32
_BPW = _B // _NW
_CH = 64


def _sc_gather_lr(y, lr2):
    mesh = plsc.VectorSubcoreMesh(core_axis_name="c", subcore_axis_name="s")

    @functools.partial(
        pl.kernel,
        mesh=mesh,
        out_type=jax.ShapeDtypeStruct((_B, _Z * _R), jnp.float32),
        scratch_types=[
            pltpu.VMEM((_BPW,), jnp.int32),
            pltpu.VMEM((_CH, _Z * _R), jnp.float32),
            pltpu.SemaphoreType.DMA,
        ],
    )
    def k(y_hbm, lr_hbm, lrg_hbm, idx_v, rows_v, sem_lr):
        wid = lax.axis_index("s") * 2 + lax.axis_index("c")
        base = wid * _BPW
        pltpu.sync_copy(y_hbm.at[pl.ds(base, _BPW)], idx_v)
        for c in range(_BPW // _CH):
            idx_c = idx_v.at[pl.ds(c * _CH, _CH)]
            pltpu.async_copy(lr_hbm.at[idx_c], rows_v, sem_lr).wait()
            pltpu.sync_copy(rows_v, lrg_hbm.at[pl.ds(base + c * _CH, _CH)])

    return k(y, lr2)


def _sc_gather_mudg(y2, mu2, dg2):
    mesh = plsc.VectorSubcoreMesh(core_axis_name="c", subcore_axis_name="s")

    @functools.partial(
        pl.kernel,
        mesh=mesh,
        out_type=(
            jax.ShapeDtypeStruct((_B, _P), jnp.float32),
            jax.ShapeDtypeStruct((_B, _P), jnp.float32),
        ),
        scratch_types=[
            pltpu.VMEM((_BPW,), jnp.int32),
            pltpu.VMEM((_BPW, _P), jnp.float32),
            pltpu.VMEM((_BPW, _P), jnp.float32),
            pltpu.SemaphoreType.DMA,
            pltpu.SemaphoreType.DMA,
        ],
    )
    def k(y2_hbm, mu_hbm, dg_hbm, mug_hbm, dgg_hbm,
          idx2_v, mu_v, dg_v, sem_mu, sem_dg):
        wid = lax.axis_index("s") * 2 + lax.axis_index("c")
        base = wid * _BPW
        pltpu.sync_copy(y2_hbm.at[pl.ds(base, _BPW)], idx2_v)
        cp_mu = pltpu.async_copy(mu_hbm.at[idx2_v], mu_v, sem_mu)
        cp_dg = pltpu.async_copy(dg_hbm.at[idx2_v], dg_v, sem_dg)
        cp_mu.wait()
        pltpu.sync_copy(mu_v, mug_hbm.at[pl.ds(base, _BPW)])
        cp_dg.wait()
        pltpu.sync_copy(dg_v, dgg_hbm.at[pl.ds(base, _BPW)])

    return k(y2, mu2, dg2)


_TC = 1024


def _tc_transpose_body(in_ref, out_ref):
    out_ref[...] = in_ref[...].T


def _tc_transpose(lrT2):
    return pl.pallas_call(
        _tc_transpose_body,
        grid=(pl.cdiv(_N, _TC),),
        in_specs=[pl.BlockSpec((_Z * _R, _TC), lambda g: (0, g))],
        out_specs=pl.BlockSpec((_TC, _Z * _R), lambda g: (g, 0)),
        out_shape=jax.ShapeDtypeStruct((_N, _Z * _R), jnp.float32),
    )(lrT2)


_BC = 256


def _tc_body(lrg_ref, mug_ref, dgg_ref, par_ref, out_ref):
    gt = lrg_ref[...].T
    gt3 = gt.reshape(_Z, _R, _BC)
    par = par_ref[...]
    mut = mug_ref[...].T
    dgt = dgg_ref[...].T
    mu_t = mut[0:_Z] + par * (mut[_Z:_P] - mut[0:_Z])
    dg_t = dgt[0:_Z] + par * (dgt[_Z:_P] - dgt[0:_Z])
    sp = jax.nn.softplus(dg_t)
    out_ref[0] = mu_t
    for j in range(_Z):
        if j > 0:
            out_ref[1 + j, 0:j] = jnp.zeros((j, _BC), jnp.float32)
        out_ref[1 + j, j:j + 1] = sp[j:j + 1]
        if j < _Z - 1:
            pj = gt3[j]
            prod = gt3[j + 1:] * pj[None]
            out_ref[1 + j, j + 1:_Z] = prod.sum(axis=1)


def _tc_build(lrg, mug, dgg, par):
    return pl.pallas_call(
        _tc_body,
        grid=(_B // _BC,),
        in_specs=[
            pl.BlockSpec((_BC, _Z * _R), lambda g: (g, 0)),
            pl.BlockSpec((_BC, _P), lambda g: (g, 0)),
            pl.BlockSpec((_BC, _P), lambda g: (g, 0)),
            pl.BlockSpec((1, _BC), lambda g: (0, g)),
        ],
        out_specs=pl.BlockSpec((_Z + 1, _Z, _BC), lambda g: (0, 0, g)),
        out_shape=jax.ShapeDtypeStruct((_Z + 1, _Z, _B), jnp.float32),
    )(lrg, mug, dgg, par)


def kernel(y, mu, low_rank, diag):
    lrT2 = jnp.transpose(low_rank, (1, 2, 0)).reshape(_Z * _R, _N)
    lr2 = _tc_transpose(lrT2)
    mu2 = mu.reshape(_N // 2, _P)
    dg2 = diag.reshape(_N // 2, _P)
    y2 = lax.div(y, 2)
    par = (y % 2).astype(jnp.float32).reshape(1, _B)
    mug, dgg = _sc_gather_mudg(y2, mu2, dg2)
    lrg = _sc_gather_lr(y, lr2)
    out_t = _tc_build(lrg, mug, dgg, par)
    return jnp.transpose(out_t, (2, 1, 0))


# speedup vs baseline: 1.1823x; 1.0143x over previous
import functools

import jax
import jax.numpy as jnp
from jax import lax
from jax.experimental import pallas as pl
from jax.experimental.pallas import tpu as pltpu
from jax.experimental.pallas import tpu_sc as plsc

_N = 100000
_Z = 64
_R = 16
_B = 4096
_P = 2 * _Z

_NW = 32
_BPW = _B // _NW
_CH = 64


def _sc_gather_lr(y, lr2):
    mesh = plsc.VectorSubcoreMesh(core_axis_name="c", subcore_axis_name="s")

    @functools.partial(
        pl.kernel,
        mesh=mesh,
        out_type=jax.ShapeDtypeStruct((_B, _Z * _R), jnp.float32),
        scratch_types=[
            pltpu.VMEM((_BPW,), jnp.int32),
            pltpu.VMEM((_CH, _Z * _R), jnp.float32),
            pltpu.SemaphoreType.DMA,
        ],
    )
    def k(y_hbm, lr_hbm, lrg_hbm, idx_v, rows_v, sem_lr):
        wid = lax.axis_index("s") * 2 + lax.axis_index("c")
        base = wid * _BPW
        pltpu.sync_copy(y_hbm.at[pl.ds(base, _BPW)], idx_v)
        for c in range(_BPW // _CH):
            idx_c = idx_v.at[pl.ds(c * _CH, _CH)]
            pltpu.async_copy(lr_hbm.at[idx_c], rows_v, sem_lr).wait()
            pltpu.sync_copy(rows_v, lrg_hbm.at[pl.ds(base + c * _CH, _CH)])

    return k(y, lr2)


def _sc_gather_mudg(y2, mu2, dg2):
    mesh = plsc.VectorSubcoreMesh(core_axis_name="c", subcore_axis_name="s")

    @functools.partial(
        pl.kernel,
        mesh=mesh,
        out_type=(
            jax.ShapeDtypeStruct((_B, _P), jnp.float32),
            jax.ShapeDtypeStruct((_B, _P), jnp.float32),
        ),
        scratch_types=[
            pltpu.VMEM((_BPW,), jnp.int32),
            pltpu.VMEM((_BPW, _P), jnp.float32),
            pltpu.VMEM((_BPW, _P), jnp.float32),
            pltpu.SemaphoreType.DMA,
            pltpu.SemaphoreType.DMA,
        ],
    )
    def k(y2_hbm, mu_hbm, dg_hbm, mug_hbm, dgg_hbm,
          idx2_v, mu_v, dg_v, sem_mu, sem_dg):
        wid = lax.axis_index("s") * 2 + lax.axis_index("c")
        base = wid * _BPW
        pltpu.sync_copy(y2_hbm.at[pl.ds(base, _BPW)], idx2_v)
        cp_mu = pltpu.async_copy(mu_hbm.at[idx2_v], mu_v, sem_mu)
        cp_dg = pltpu.async_copy(dg_hbm.at[idx2_v], dg_v, sem_dg)
        cp_mu.wait()
        pltpu.sync_copy(mu_v, mug_hbm.at[pl.ds(base, _BPW)])
        cp_dg.wait()
        pltpu.sync_copy(dg_v, dgg_hbm.at[pl.ds(base, _BPW)])

    return k(y2, mu2, dg2)


_TC = 2048


def _tc_transpose_body(in_ref, out_ref):
    out_ref[...] = in_ref[...].T


def _tc_transpose(lrT2):
    return pl.pallas_call(
        _tc_transpose_body,
        grid=(pl.cdiv(_N, _TC),),
        in_specs=[pl.BlockSpec((_Z * _R, _TC), lambda g: (0, g))],
        out_specs=pl.BlockSpec((_TC, _Z * _R), lambda g: (g, 0)),
        out_shape=jax.ShapeDtypeStruct((_N, _Z * _R), jnp.float32),
    )(lrT2)


_BC = 256


def _tc_body(lrg_ref, mug_ref, dgg_ref, par_ref, out_ref):
    gt = lrg_ref[...].T
    gt3 = gt.reshape(_Z, _R, _BC)
    par = par_ref[...]
    mut = mug_ref[...].T
    dgt = dgg_ref[...].T
    mu_t = mut[0:_Z] + par * (mut[_Z:_P] - mut[0:_Z])
    dg_t = dgt[0:_Z] + par * (dgt[_Z:_P] - dgt[0:_Z])
    sp = jax.nn.softplus(dg_t)
    out_ref[0] = mu_t
    for j in range(_Z):
        if j > 0:
            out_ref[1 + j, 0:j] = jnp.zeros((j, _BC), jnp.float32)
        out_ref[1 + j, j:j + 1] = sp[j:j + 1]
        if j < _Z - 1:
            pj = gt3[j]
            prod = gt3[j + 1:] * pj[None]
            out_ref[1 + j, j + 1:_Z] = prod.sum(axis=1)


def _tc_build(lrg, mug, dgg, par):
    return pl.pallas_call(
        _tc_body,
        grid=(_B // _BC,),
        in_specs=[
            pl.BlockSpec((_BC, _Z * _R), lambda g: (g, 0)),
            pl.BlockSpec((_BC, _P), lambda g: (g, 0)),
            pl.BlockSpec((_BC, _P), lambda g: (g, 0)),
            pl.BlockSpec((1, _BC), lambda g: (0, g)),
        ],
        out_specs=pl.BlockSpec((_Z + 1, _Z, _BC), lambda g: (0, 0, g)),
        out_shape=jax.ShapeDtypeStruct((_Z + 1, _Z, _B), jnp.float32),
    )(lrg, mug, dgg, par)


def kernel(y, mu, low_rank, diag):
    lrT2 = jnp.transpose(low_rank, (1, 2, 0)).reshape(_Z * _R, _N)
    lr2 = _tc_transpose(lrT2)
    mu2 = mu.reshape(_N // 2, _P)
    dg2 = diag.reshape(_N // 2, _P)
    y2 = lax.div(y, 2)
    par = (y % 2).astype(jnp.float32).reshape(1, _B)
    mug, dgg = _sc_gather_mudg(y2, mu2, dg2)
    lrg = _sc_gather_lr(y, lr2)
    out_t = _tc_build(lrg, mug, dgg, par)
    return jnp.transpose(out_t, (2, 1, 0))
